# baseline (speedup 1.0000x reference)
import functools

import jax
import jax.numpy as jnp
from jax import lax
from jax.experimental import pallas as pl
from jax.experimental.pallas import tpu as pltpu
from jax.experimental.pallas import tpu_sc as plsc

_NUM_WORKERS = 32
_GRP = 8


@functools.partial(jax.jit, static_argnums=(2, 3, 4))
def _embed_lookup(sentence, table, b, s, d):
    sents_per_w = b // _NUM_WORKERS
    n_grps = sents_per_w // _GRP
    mesh = plsc.VectorSubcoreMesh(core_axis_name="c", subcore_axis_name="s")

    @functools.partial(
        pl.kernel,
        mesh=mesh,
        out_type=jax.ShapeDtypeStruct((b, s, d), jnp.float32),
        scratch_types=[
            pltpu.VMEM((_GRP, s), jnp.int32),
            pltpu.SemaphoreType.DMA,
            pltpu.SemaphoreType.DMA,
        ],
    )
    def k(sent_hbm, table_hbm, out_hbm, idx_v, sem_g, sem_i):
        wid = lax.axis_index("s") * 2 + lax.axis_index("c")
        base = wid * sents_per_w

        def grp_body(g, _):
            s0 = base + g * _GRP
            pltpu.sync_copy(sent_hbm.at[pl.ds(s0, _GRP), :], idx_v)
            for j in range(_GRP):
                sent = s0 + j

                def issue16(off, lanes):
                    v = idx_v[j, pl.ds(off, 16)]
                    for l in lanes:
                        pltpu.async_copy(
                            table_hbm.at[pl.ds(v[l], 1), :],
                            out_hbm.at[sent, pl.ds(off + l, 1), :],
                            sem_g)

                def blk_body(t16, _):
                    issue16(t16 * 16, range(16))
                    return 0
                lax.fori_loop(0, s // 16, blk_body, 0)
                if s % 16:
                    issue16(s - 16, range(16 - s % 16, 16))
            return 0

        lax.fori_loop(0, n_grps, grp_body, 0)

        pltpu.make_async_copy(
            out_hbm.at[pl.ds(base, sents_per_w), :, :],
            out_hbm.at[pl.ds(base, sents_per_w), :, :],
            sem_g).wait()

    return k(sentence, table)


def kernel(sentence, table):
    b, s = sentence.shape
    v, d = table.shape
    return _embed_lookup(sentence.astype(jnp.int32), table, b, s, d)


# speedup vs baseline: 14.2136x; 14.2136x over previous
import functools

import jax
import jax.numpy as jnp
from jax import lax
from jax.experimental import pallas as pl
from jax.experimental.pallas import tpu as pltpu
from jax.experimental.pallas import tpu_sc as plsc

_NUM_WORKERS = 32
_GRP = 8


@functools.partial(jax.jit, static_argnums=(2, 3, 4))
def _embed_lookup(sentence, table, b, s, d):
    sents_per_w = b // _NUM_WORKERS
    n_grps = sents_per_w // _GRP
    mesh = plsc.VectorSubcoreMesh(core_axis_name="c", subcore_axis_name="s")

    @functools.partial(
        pl.kernel,
        mesh=mesh,
        out_type=jax.ShapeDtypeStruct((b, s, d), jnp.float32),
        scratch_types=(
            [pltpu.VMEM((_GRP, s), jnp.int32)]
            + [pltpu.VMEM((s, d), jnp.float32) for _ in range(2)]
            + [pltpu.SemaphoreType.DMA for _ in range(4)]
        ),
    )
    def k(sent_hbm, table_hbm, out_hbm, idx_v, rows0, rows1,
          gs0, gs1, ss0, ss1):
        rows_v = (rows0, rows1)
        sem_g = (gs0, gs1)
        sem_s = (ss0, ss1)

        wid = lax.axis_index("s") * 2 + lax.axis_index("c")
        base = wid * sents_per_w

        def store_wait(bf):
            pltpu.make_async_copy(
                rows_v[bf], out_hbm.at[base], sem_s[bf]).wait()

        def gather_drain(bf):
            pltpu.make_async_copy(
                out_hbm.at[base], rows_v[bf], sem_g[bf]).wait()

        def grp_body(g, _):
            s0 = base + g * _GRP
            pltpu.sync_copy(sent_hbm.at[pl.ds(s0, _GRP), :], idx_v)
            for j in range(_GRP):
                bf = j % 2

                if j >= 2:
                    store_wait(bf)
                else:
                    @pl.when(g > 0)
                    def _drain_prev():
                        store_wait(bf)

                def issue16(off, lanes):
                    v = idx_v[j, pl.ds(off, 16)]
                    for l in lanes:
                        pltpu.async_copy(
                            table_hbm.at[pl.ds(v[l], 1), :],
                            rows_v[bf].at[pl.ds(off + l, 1), :],
                            sem_g[bf])

                def blk_body(t16, _):
                    issue16(t16 * 16, range(16))
                    return 0
                lax.fori_loop(0, s // 16, blk_body, 0)
                if s % 16:
                    issue16(s - 16, range(16 - s % 16, 16))

                if j > 0:
                    pb = (j - 1) % 2
                    gather_drain(pb)
                    pltpu.async_copy(
                        rows_v[pb], out_hbm.at[s0 + j - 1], sem_s[pb])
            lb = (_GRP - 1) % 2
            gather_drain(lb)
            pltpu.async_copy(
                rows_v[lb], out_hbm.at[s0 + _GRP - 1], sem_s[lb])
            return 0

        lax.fori_loop(0, n_grps, grp_body, 0)

        for bf in range(2):
            store_wait(bf)

    return k(sentence, table)


def kernel(sentence, table):
    b, s = sentence.shape
    v, d = table.shape
    return _embed_lookup(sentence.astype(jnp.int32), table, b, s, d)


# speedup vs baseline: 14.2278x; 1.0010x over previous
import functools

import jax
import jax.numpy as jnp
from jax import lax
from jax.experimental import pallas as pl
from jax.experimental.pallas import tpu as pltpu
from jax.experimental.pallas import tpu_sc as plsc

_NUM_WORKERS = 32
_GRP = 8
_NROW = 4


@functools.partial(jax.jit, static_argnums=(2, 3, 4))
def _embed_lookup(sentence, table, b, s, d):
    sents_per_w = b // _NUM_WORKERS
    n_grps = sents_per_w // _GRP
    n_outer = n_grps // 2
    mesh = plsc.VectorSubcoreMesh(core_axis_name="c", subcore_axis_name="s")

    @functools.partial(
        pl.kernel,
        mesh=mesh,
        out_type=jax.ShapeDtypeStruct((b, s, d), jnp.float32),
        scratch_types=(
            [pltpu.VMEM((_GRP, s), jnp.int32) for _ in range(2)]
            + [pltpu.VMEM((s, d), jnp.float32) for _ in range(_NROW)]
            + [pltpu.SemaphoreType.DMA for _ in range(2 + 2 * _NROW)]
        ),
    )
    def k(sent_hbm, table_hbm, out_hbm, *scr):
        idx_v = scr[:2]
        rows_v = scr[2:2 + _NROW]
        sem_i = scr[2 + _NROW:4 + _NROW]
        sem_g = scr[4 + _NROW:4 + 2 * _NROW]
        sem_s = scr[4 + 2 * _NROW:4 + 3 * _NROW]

        wid = lax.axis_index("s") * 2 + lax.axis_index("c")
        base = wid * sents_per_w

        def idx_start(grp, ib):
            pltpu.async_copy(
                sent_hbm.at[pl.ds(base + grp * _GRP, _GRP), :],
                idx_v[ib], sem_i[ib])

        def idx_wait(ib):
            pltpu.make_async_copy(
                sent_hbm.at[pl.ds(base, _GRP), :], idx_v[ib],
                sem_i[ib]).wait()

        def store_wait(bf):
            pltpu.make_async_copy(
                rows_v[bf], out_hbm.at[base], sem_s[bf]).wait()

        def gather_drain(bf):
            pltpu.make_async_copy(
                out_hbm.at[base], rows_v[bf], sem_g[bf]).wait()

        def process_group(grp, ib, first_cond):
            s0 = base + grp * _GRP
            for j in range(_GRP):
                bf = j % _NROW

                if j >= _NROW:
                    store_wait(bf)
                else:
                    @pl.when(jnp.logical_not(first_cond))
                    def _drain_prev():
                        store_wait(bf)

                def issue16(off, lanes):
                    v = idx_v[ib][j, pl.ds(off, 16)]
                    for l in lanes:
                        pltpu.async_copy(
                            table_hbm.at[pl.ds(v[l], 1), :],
                            rows_v[bf].at[pl.ds(off + l, 1), :],
                            sem_g[bf])

                def blk_body(t16, _):
                    issue16(t16 * 16, range(16))
                    return 0
                lax.fori_loop(0, s // 16, blk_body, 0)
                if s % 16:
                    issue16(s - 16, range(16 - s % 16, 16))

                if j > 0:
                    pb = (j - 1) % _NROW
                    gather_drain(pb)
                    pltpu.async_copy(
                        rows_v[pb], out_hbm.at[s0 + j - 1], sem_s[pb])
            lb = (_GRP - 1) % _NROW
            gather_drain(lb)
            pltpu.async_copy(
                rows_v[lb], out_hbm.at[s0 + _GRP - 1], sem_s[lb])

        idx_start(0, 0)

        def outer(g2, _):
            idx_wait(0)
            idx_start(2 * g2 + 1, 1)
            process_group(2 * g2, 0, g2 == 0)
            idx_wait(1)

            @pl.when(g2 < n_outer - 1)
            def _prefetch():
                idx_start(2 * g2 + 2, 0)
            process_group(2 * g2 + 1, 1, g2 < 0)
            return 0

        lax.fori_loop(0, n_outer, outer, 0)

        for bf in range(min(_NROW, _GRP)):
            store_wait(bf)

    return k(sentence, table)


def kernel(sentence, table):
    b, s = sentence.shape
    v, d = table.shape
    return _embed_lookup(sentence.astype(jnp.int32), table, b, s, d)


# speedup vs baseline: 16.0281x; 1.1265x over previous
import functools

import jax
import jax.numpy as jnp
from jax import lax
from jax.experimental import pallas as pl
from jax.experimental.pallas import tpu as pltpu
from jax.experimental.pallas import tpu_sc as plsc

_NUM_WORKERS = 32
_GRP = 8
_NROW = 4


@functools.partial(jax.jit, static_argnums=(2, 3, 4))
def _embed_lookup(sentence, table, b, s, d):
    sents_per_w = b // _NUM_WORKERS
    n_grps = sents_per_w // _GRP
    n_outer = n_grps // 2
    mesh = plsc.VectorSubcoreMesh(core_axis_name="c", subcore_axis_name="s")

    @functools.partial(
        pl.kernel,
        mesh=mesh,
        out_type=jax.ShapeDtypeStruct((b * s, d), jnp.float32),
        scratch_types=(
            [pltpu.VMEM((_GRP, s), jnp.int32) for _ in range(2)]
            + [pltpu.VMEM((s, d), jnp.float32) for _ in range(_NROW)]
            + [pltpu.SemaphoreType.DMA for _ in range(2 + 2 * _NROW)]
        ),
    )
    def k(sent_hbm, table_hbm, out_hbm, *scr):
        idx_v = scr[:2]
        rows_v = scr[2:2 + _NROW]
        sem_i = scr[2 + _NROW:4 + _NROW]
        sem_g = scr[4 + _NROW:4 + 2 * _NROW]
        sem_s = scr[4 + 2 * _NROW:4 + 3 * _NROW]

        wid = lax.axis_index("s") * 2 + lax.axis_index("c")
        base = wid * sents_per_w

        def idx_start(grp, ib):
            pltpu.async_copy(
                sent_hbm.at[pl.ds(base + grp * _GRP, _GRP), :],
                idx_v[ib], sem_i[ib])

        def idx_wait(ib):
            pltpu.make_async_copy(
                sent_hbm.at[pl.ds(base, _GRP), :], idx_v[ib],
                sem_i[ib]).wait()

        def store_wait(bf):
            pltpu.make_async_copy(
                rows_v[bf], out_hbm.at[pl.ds(base * s, s), :],
                sem_s[bf]).wait()

        def gather_drain(bf):
            pltpu.make_async_copy(
                out_hbm.at[pl.ds(base * s, s), :], rows_v[bf],
                sem_g[bf]).wait()

        def process_group(grp, ib, first_cond):
            s0 = base + grp * _GRP
            for j in range(_GRP):
                bf = j % _NROW

                if j >= _NROW:
                    store_wait(bf)
                else:
                    @pl.when(jnp.logical_not(first_cond))
                    def _drain_prev():
                        store_wait(bf)

                def issue16(off, lanes):
                    v = idx_v[ib][j, pl.ds(off, 16)]
                    for l in lanes:
                        pltpu.async_copy(
                            table_hbm.at[pl.ds(v[l], 1), :],
                            rows_v[bf].at[pl.ds(off + l, 1), :],
                            sem_g[bf])

                def blk_body(t16, _):
                    issue16(t16 * 16, range(16))
                    return 0
                lax.fori_loop(0, s // 16, blk_body, 0)
                if s % 16:
                    issue16(s - 16, range(16 - s % 16, 16))

                if j > 0:
                    pb = (j - 1) % _NROW
                    gather_drain(pb)
                    pltpu.async_copy(
                        rows_v[pb],
                        out_hbm.at[pl.ds((s0 + j - 1) * s, s), :],
                        sem_s[pb])
            lb = (_GRP - 1) % _NROW
            gather_drain(lb)
            pltpu.async_copy(
                rows_v[lb],
                out_hbm.at[pl.ds((s0 + _GRP - 1) * s, s), :], sem_s[lb])

        idx_start(0, 0)

        def outer(g2, _):
            idx_wait(0)
            idx_start(2 * g2 + 1, 1)
            process_group(2 * g2, 0, g2 == 0)
            idx_wait(1)

            @pl.when(g2 < n_outer - 1)
            def _prefetch():
                idx_start(2 * g2 + 2, 0)
            process_group(2 * g2 + 1, 1, g2 < 0)
            return 0

        lax.fori_loop(0, n_outer, outer, 0)

        for bf in range(min(_NROW, _GRP)):
            store_wait(bf)

    return k(sentence, table)


def kernel(sentence, table):
    b, s = sentence.shape
    v, d = table.shape
    out2 = _embed_lookup(sentence.astype(jnp.int32), table, b, s, d)
    return out2.reshape(b, s, d)
